# Initial kernel scaffold; baseline (speedup 1.0000x reference)
import jax
import jax.numpy as jnp
from jax.experimental import pallas as pl


def kernel(x, edge_index, batch, W1, a_src1, a_dst1, b1, W2, a_src2, a_dst2, b2, W3, a_src3, a_dst3, b3, Wl, bl):
    raise NotImplementedError("write your pallas kernel here")


# speedup vs baseline: 4.2022x; 4.2022x over previous
import jax
import jax.numpy as jnp
from jax.experimental import pallas as pl

_N = 10000
_E = 160000
_D = 256
_H = 4
_C = 256
_HC = _H * _C
_G = 64
_NC = 10

_NB = 1000
_EB = 1000
_E2 = _E + _N


def _proj_kernel(x_ref, w_ref, m_ref, h_ref, s_ref):
    h = jnp.dot(x_ref[...], w_ref[...], preferred_element_type=jnp.float32)
    h_ref[...] = h
    s_ref[...] = jnp.dot(h, m_ref[...], preferred_element_type=jnp.float32)


def _edge_e_kernel(es_ref, ed_ref, e_ref):
    s = es_ref[...] + ed_ref[...]
    e_ref[...] = jnp.where(s >= 0, s, 0.2 * s)


def _edge_ex_kernel(e_ref, emaxd_ref, ex_ref):
    ex_ref[...] = jnp.exp(e_ref[...] - emaxd_ref[...])


def _edge_msg_kernel(hs_ref, ex_ref, den_ref, r_ref, msg_ref):
    alpha = ex_ref[...] / (den_ref[...] + 1e-16)
    arep = jnp.dot(alpha, r_ref[...], preferred_element_type=jnp.float32)
    msg_ref[...] = hs_ref[...] * arep


def _final_kernel(p_ref, wl_ref, bl_ref, o_ref):
    o_ref[...] = (
        jnp.dot(p_ref[...], wl_ref[...], preferred_element_type=jnp.float32)
        + bl_ref[...]
    )


def _project(x, W, Msd):
    fin = x.shape[1]
    return pl.pallas_call(
        _proj_kernel,
        grid=(_N // _NB,),
        in_specs=[
            pl.BlockSpec((_NB, fin), lambda i: (i, 0)),
            pl.BlockSpec((fin, _HC), lambda i: (0, 0)),
            pl.BlockSpec((_HC, 2 * _H), lambda i: (0, 0)),
        ],
        out_specs=[
            pl.BlockSpec((_NB, _HC), lambda i: (i, 0)),
            pl.BlockSpec((_NB, 2 * _H), lambda i: (i, 0)),
        ],
        out_shape=[
            jax.ShapeDtypeStruct((_N, _HC), jnp.float32),
            jax.ShapeDtypeStruct((_N, 2 * _H), jnp.float32),
        ],
    )(x, W, Msd)


def _edge_e(es, ed):
    return pl.pallas_call(
        _edge_e_kernel,
        grid=(_E2 // _EB,),
        in_specs=[
            pl.BlockSpec((_EB, _H), lambda i: (i, 0)),
            pl.BlockSpec((_EB, _H), lambda i: (i, 0)),
        ],
        out_specs=pl.BlockSpec((_EB, _H), lambda i: (i, 0)),
        out_shape=jax.ShapeDtypeStruct((_E2, _H), jnp.float32),
    )(es, ed)


def _edge_ex(e, emaxd):
    return pl.pallas_call(
        _edge_ex_kernel,
        grid=(_E2 // _EB,),
        in_specs=[
            pl.BlockSpec((_EB, _H), lambda i: (i, 0)),
            pl.BlockSpec((_EB, _H), lambda i: (i, 0)),
        ],
        out_specs=pl.BlockSpec((_EB, _H), lambda i: (i, 0)),
        out_shape=jax.ShapeDtypeStruct((_E2, _H), jnp.float32),
    )(e, emaxd)


def _edge_msg(hs, ex, den, R):
    return pl.pallas_call(
        _edge_msg_kernel,
        grid=(_E2 // _EB,),
        in_specs=[
            pl.BlockSpec((_EB, _HC), lambda i: (i, 0)),
            pl.BlockSpec((_EB, _H), lambda i: (i, 0)),
            pl.BlockSpec((_EB, _H), lambda i: (i, 0)),
            pl.BlockSpec((_H, _HC), lambda i: (0, 0)),
        ],
        out_specs=pl.BlockSpec((_EB, _HC), lambda i: (i, 0)),
        out_shape=jax.ShapeDtypeStruct((_E2, _HC), jnp.float32),
    )(hs, ex, den, R)


def _final(pooled, Wl, bl):
    return pl.pallas_call(
        _final_kernel,
        out_shape=jax.ShapeDtypeStruct((_G, _NC), jnp.float32),
    )(pooled, Wl, bl.reshape(1, _NC))


def _gat_layer(x, src, dst, W, a_src, a_dst, b, R):
    eyeH = jnp.eye(_H, dtype=jnp.float32)
    Msrc = (a_src[:, :, None] * eyeH[:, None, :]).reshape(_HC, _H)
    Mdst = (a_dst[:, :, None] * eyeH[:, None, :]).reshape(_HC, _H)
    Msd = jnp.concatenate([Msrc, Mdst], axis=1)

    h, s = _project(x, W, Msd)
    es = jnp.take(s, src, axis=0)[:, :_H]
    ed = jnp.take(s, dst, axis=0)[:, _H:]
    e = _edge_e(es, ed)
    emax = jax.ops.segment_max(e, dst, num_segments=_N)
    emax = jnp.where(jnp.isfinite(emax), emax, 0.0)
    ex = _edge_ex(e, jnp.take(emax, dst, axis=0))
    denom = jax.ops.segment_sum(ex, dst, num_segments=_N)
    hs = jnp.take(h, src, axis=0)
    msg = _edge_msg(hs, ex, jnp.take(denom, dst, axis=0), R)
    out = jax.ops.segment_sum(msg, dst, num_segments=_N)
    return out + b


def kernel(x, edge_index, batch, W1, a_src1, a_dst1, b1, W2, a_src2, a_dst2, b2,
           W3, a_src3, a_dst3, b3, Wl, bl):
    loop = jnp.arange(_N, dtype=edge_index.dtype)
    src = jnp.concatenate([edge_index[0], loop])
    dst = jnp.concatenate([edge_index[1], loop])
    R = jnp.kron(jnp.eye(_H, dtype=jnp.float32), jnp.ones((1, _C), jnp.float32))

    h = jax.nn.relu(_gat_layer(x, src, dst, W1, a_src1, a_dst1, b1, R))
    h = jax.nn.relu(_gat_layer(h, src, dst, W2, a_src2, a_dst2, b2, R))
    h = _gat_layer(h, src, dst, W3, a_src3, a_dst3, b3, R)

    sums = jax.ops.segment_sum(h, batch, num_segments=_G)
    counts = jax.ops.segment_sum(jnp.ones((_N, 1), jnp.float32), batch,
                                 num_segments=_G)
    pooled = sums / jnp.maximum(counts, 1.0)
    return _final(pooled, Wl, bl)


# speedup vs baseline: 4.2693x; 1.0160x over previous
import jax
import jax.numpy as jnp
from jax.experimental import pallas as pl

_N = 10000
_E = 160000
_D = 256
_H = 4
_C = 256
_HC = _H * _C
_G = 64
_NC = 10

_NB = 1000
_EB = 1000
_E2 = _E + _N


def _proj_kernel(x_ref, w_ref, m_ref, h_ref, s_ref):
    h = jnp.dot(x_ref[...], w_ref[...], preferred_element_type=jnp.float32)
    h_ref[...] = h
    s_ref[...] = jnp.dot(h, m_ref[...], preferred_element_type=jnp.float32)


def _edge_e_kernel(es_ref, ed_ref, e_ref):
    s = es_ref[...] + ed_ref[...]
    e_ref[...] = jnp.where(s >= 0, s, 0.2 * s)


def _edge_ex_kernel(e_ref, emaxd_ref, ex_ref):
    ex_ref[...] = jnp.exp(e_ref[...] - emaxd_ref[...])


def _edge_msg_kernel(hs_ref, ex_ref, den_ref, r_ref, msg_ref):
    alpha = ex_ref[...] / (den_ref[...] + 1e-16)
    arep = jnp.dot(alpha, r_ref[...], preferred_element_type=jnp.float32)
    msg_ref[...] = hs_ref[...] * arep


def _final_kernel(p_ref, wl_ref, bl_ref, o_ref):
    o_ref[...] = (
        jnp.dot(p_ref[...], wl_ref[...], preferred_element_type=jnp.float32)
        + bl_ref[...]
    )


def _project(x, W, Msd):
    fin = x.shape[1]
    return pl.pallas_call(
        _proj_kernel,
        grid=(_N // _NB,),
        in_specs=[
            pl.BlockSpec((_NB, fin), lambda i: (i, 0)),
            pl.BlockSpec((fin, _HC), lambda i: (0, 0)),
            pl.BlockSpec((_HC, 2 * _H), lambda i: (0, 0)),
        ],
        out_specs=[
            pl.BlockSpec((_NB, _HC), lambda i: (i, 0)),
            pl.BlockSpec((_NB, 2 * _H), lambda i: (i, 0)),
        ],
        out_shape=[
            jax.ShapeDtypeStruct((_N, _HC), jnp.float32),
            jax.ShapeDtypeStruct((_N, 2 * _H), jnp.float32),
        ],
    )(x, W, Msd)


def _edge_e(es, ed):
    return pl.pallas_call(
        _edge_e_kernel,
        grid=(_E2 // _EB,),
        in_specs=[
            pl.BlockSpec((_EB, _H), lambda i: (i, 0)),
            pl.BlockSpec((_EB, _H), lambda i: (i, 0)),
        ],
        out_specs=pl.BlockSpec((_EB, _H), lambda i: (i, 0)),
        out_shape=jax.ShapeDtypeStruct((_E2, _H), jnp.float32),
    )(es, ed)


def _edge_ex(e, emaxd):
    return pl.pallas_call(
        _edge_ex_kernel,
        grid=(_E2 // _EB,),
        in_specs=[
            pl.BlockSpec((_EB, _H), lambda i: (i, 0)),
            pl.BlockSpec((_EB, _H), lambda i: (i, 0)),
        ],
        out_specs=pl.BlockSpec((_EB, _H), lambda i: (i, 0)),
        out_shape=jax.ShapeDtypeStruct((_E2, _H), jnp.float32),
    )(e, emaxd)


def _edge_msg(hs, ex, den, R):
    return pl.pallas_call(
        _edge_msg_kernel,
        grid=(_E2 // _EB,),
        in_specs=[
            pl.BlockSpec((_EB, _HC), lambda i: (i, 0)),
            pl.BlockSpec((_EB, _H), lambda i: (i, 0)),
            pl.BlockSpec((_EB, _H), lambda i: (i, 0)),
            pl.BlockSpec((_H, _HC), lambda i: (0, 0)),
        ],
        out_specs=pl.BlockSpec((_EB, _HC), lambda i: (i, 0)),
        out_shape=jax.ShapeDtypeStruct((_E2, _HC), jnp.float32),
    )(hs, ex, den, R)


def _final(pooled, Wl, bl):
    return pl.pallas_call(
        _final_kernel,
        out_shape=jax.ShapeDtypeStruct((_G, _NC), jnp.float32),
    )(pooled, Wl, bl.reshape(1, _NC))


def _gat_layer(x, src, dst, W, a_src, a_dst, b, R):
    eyeH = jnp.eye(_H, dtype=jnp.float32)
    Msrc = (a_src[:, :, None] * eyeH[:, None, :]).reshape(_HC, _H)
    Mdst = (a_dst[:, :, None] * eyeH[:, None, :]).reshape(_HC, _H)
    Msd = jnp.concatenate([Msrc, Mdst], axis=1)

    h, s = _project(x, W, Msd)
    es = jnp.take(s, src, axis=0)[:, :_H]
    ed = jnp.take(s, dst, axis=0)[:, _H:]
    e = _edge_e(es, ed)
    emax = jax.ops.segment_max(e, dst, num_segments=_N,
                               indices_are_sorted=True)
    emax = jnp.where(jnp.isfinite(emax), emax, 0.0)
    ex = _edge_ex(e, jnp.take(emax, dst, axis=0))
    denom = jax.ops.segment_sum(ex, dst, num_segments=_N,
                                indices_are_sorted=True)
    hs = jnp.take(h, src, axis=0)
    msg = _edge_msg(hs, ex, jnp.take(denom, dst, axis=0), R)
    out = jax.ops.segment_sum(msg, dst, num_segments=_N,
                              indices_are_sorted=True)
    return out + b


def kernel(x, edge_index, batch, W1, a_src1, a_dst1, b1, W2, a_src2, a_dst2, b2,
           W3, a_src3, a_dst3, b3, Wl, bl):
    loop = jnp.arange(_N, dtype=edge_index.dtype)
    src = jnp.concatenate([edge_index[0], loop])
    dst = jnp.concatenate([edge_index[1], loop])
    perm = jnp.argsort(dst)
    src = jnp.take(src, perm, axis=0)
    dst = jnp.take(dst, perm, axis=0)
    R = jnp.kron(jnp.eye(_H, dtype=jnp.float32), jnp.ones((1, _C), jnp.float32))

    h = jax.nn.relu(_gat_layer(x, src, dst, W1, a_src1, a_dst1, b1, R))
    h = jax.nn.relu(_gat_layer(h, src, dst, W2, a_src2, a_dst2, b2, R))
    h = _gat_layer(h, src, dst, W3, a_src3, a_dst3, b3, R)

    sums = jax.ops.segment_sum(h, batch, num_segments=_G)
    counts = jax.ops.segment_sum(jnp.ones((_N, 1), jnp.float32), batch,
                                 num_segments=_G)
    pooled = sums / jnp.maximum(counts, 1.0)
    return _final(pooled, Wl, bl)


# speedup vs baseline: 4.3387x; 1.0163x over previous
import jax
import jax.numpy as jnp
from jax.experimental import pallas as pl

_N = 10000
_E = 160000
_D = 256
_H = 4
_C = 256
_HC = _H * _C
_G = 64
_NC = 10

_NB = 1000
_EB = 2000
_E2 = _E + _N


def _proj_kernel(x_ref, w_ref, m_ref, h_ref, s_ref):
    h = jnp.dot(x_ref[...], w_ref[...], preferred_element_type=jnp.float32)
    h_ref[...] = h
    s_ref[...] = jnp.dot(h, m_ref[...], preferred_element_type=jnp.float32)


def _edge_e_kernel(es_ref, ed_ref, e_ref):
    s = es_ref[...] + ed_ref[...]
    e_ref[...] = jnp.where(s >= 0, s, 0.2 * s)


def _edge_ex_kernel(e_ref, emaxd_ref, ex_ref):
    ex_ref[...] = jnp.exp(e_ref[...] - emaxd_ref[...])


def _edge_msg_kernel(hs_ref, ex_ref, den_ref, r_ref, msg_ref):
    alpha = ex_ref[...] / (den_ref[...] + 1e-16)
    arep = jnp.dot(alpha, r_ref[...], preferred_element_type=jnp.float32)
    msg_ref[...] = hs_ref[...] * arep


def _final_kernel(p_ref, wl_ref, bl_ref, o_ref):
    o_ref[...] = (
        jnp.dot(p_ref[...], wl_ref[...], preferred_element_type=jnp.float32)
        + bl_ref[...]
    )


def _project(x, W, Msd):
    fin = x.shape[1]
    return pl.pallas_call(
        _proj_kernel,
        grid=(_N // _NB,),
        in_specs=[
            pl.BlockSpec((_NB, fin), lambda i: (i, 0)),
            pl.BlockSpec((fin, _HC), lambda i: (0, 0)),
            pl.BlockSpec((_HC, 2 * _H), lambda i: (0, 0)),
        ],
        out_specs=[
            pl.BlockSpec((_NB, _HC), lambda i: (i, 0)),
            pl.BlockSpec((_NB, 2 * _H), lambda i: (i, 0)),
        ],
        out_shape=[
            jax.ShapeDtypeStruct((_N, _HC), jnp.float32),
            jax.ShapeDtypeStruct((_N, 2 * _H), jnp.float32),
        ],
    )(x, W, Msd)


def _edge_e(es, ed):
    return pl.pallas_call(
        _edge_e_kernel,
        grid=(_E2 // _EB,),
        in_specs=[
            pl.BlockSpec((_EB, _H), lambda i: (i, 0)),
            pl.BlockSpec((_EB, _H), lambda i: (i, 0)),
        ],
        out_specs=pl.BlockSpec((_EB, _H), lambda i: (i, 0)),
        out_shape=jax.ShapeDtypeStruct((_E2, _H), jnp.float32),
    )(es, ed)


def _edge_ex(e, emaxd):
    return pl.pallas_call(
        _edge_ex_kernel,
        grid=(_E2 // _EB,),
        in_specs=[
            pl.BlockSpec((_EB, _H), lambda i: (i, 0)),
            pl.BlockSpec((_EB, _H), lambda i: (i, 0)),
        ],
        out_specs=pl.BlockSpec((_EB, _H), lambda i: (i, 0)),
        out_shape=jax.ShapeDtypeStruct((_E2, _H), jnp.float32),
    )(e, emaxd)


def _edge_msg(hs, ex, den, R):
    return pl.pallas_call(
        _edge_msg_kernel,
        grid=(_E2 // _EB,),
        in_specs=[
            pl.BlockSpec((_EB, _HC), lambda i: (i, 0)),
            pl.BlockSpec((_EB, _H), lambda i: (i, 0)),
            pl.BlockSpec((_EB, _H), lambda i: (i, 0)),
            pl.BlockSpec((_H, _HC), lambda i: (0, 0)),
        ],
        out_specs=pl.BlockSpec((_EB, _HC), lambda i: (i, 0)),
        out_shape=jax.ShapeDtypeStruct((_E2, _HC), jnp.float32),
    )(hs, ex, den, R)


def _final(pooled, Wl, bl):
    return pl.pallas_call(
        _final_kernel,
        out_shape=jax.ShapeDtypeStruct((_G, _NC), jnp.float32),
    )(pooled, Wl, bl.reshape(1, _NC))


def _gat_layer(x, src, dst, W, a_src, a_dst, b, R):
    eyeH = jnp.eye(_H, dtype=jnp.float32)
    Msrc = (a_src[:, :, None] * eyeH[:, None, :]).reshape(_HC, _H)
    Mdst = (a_dst[:, :, None] * eyeH[:, None, :]).reshape(_HC, _H)
    Msd = jnp.concatenate([Msrc, Mdst], axis=1)

    h, s = _project(x, W, Msd)
    es = jnp.take(s, src, axis=0)[:, :_H]
    ed = jnp.take(s, dst, axis=0)[:, _H:]
    e = _edge_e(es, ed)
    emax = jax.ops.segment_max(e, dst, num_segments=_N,
                               indices_are_sorted=True)
    emax = jnp.where(jnp.isfinite(emax), emax, 0.0)
    ex = _edge_ex(e, jnp.take(emax, dst, axis=0))
    denom = jax.ops.segment_sum(ex, dst, num_segments=_N,
                                indices_are_sorted=True)
    hs = jnp.take(h, src, axis=0)
    msg = _edge_msg(hs, ex, jnp.take(denom, dst, axis=0), R)
    out = jax.ops.segment_sum(msg, dst, num_segments=_N,
                              indices_are_sorted=True)
    return out + b


def kernel(x, edge_index, batch, W1, a_src1, a_dst1, b1, W2, a_src2, a_dst2, b2,
           W3, a_src3, a_dst3, b3, Wl, bl):
    loop = jnp.arange(_N, dtype=edge_index.dtype)
    src = jnp.concatenate([edge_index[0], loop])
    dst = jnp.concatenate([edge_index[1], loop])
    perm = jnp.argsort(dst)
    src = jnp.take(src, perm, axis=0)
    dst = jnp.take(dst, perm, axis=0)
    R = jnp.kron(jnp.eye(_H, dtype=jnp.float32), jnp.ones((1, _C), jnp.float32))

    h = jax.nn.relu(_gat_layer(x, src, dst, W1, a_src1, a_dst1, b1, R))
    h = jax.nn.relu(_gat_layer(h, src, dst, W2, a_src2, a_dst2, b2, R))
    h = _gat_layer(h, src, dst, W3, a_src3, a_dst3, b3, R)

    sums = jax.ops.segment_sum(h, batch, num_segments=_G)
    counts = jax.ops.segment_sum(jnp.ones((_N, 1), jnp.float32), batch,
                                 num_segments=_G)
    pooled = sums / jnp.maximum(counts, 1.0)
    return _final(pooled, Wl, bl)
